# baseline (device time: 11537 ns/iter reference)
import jax
import jax.numpy as jnp
from jax import lax
from jax.experimental import pallas as pl
from jax.experimental.pallas import tpu as pltpu

N_DEV = 16


def kernel(x, dy, gamma):
    del gamma
    m, d = x.shape

    def body(
        x_hbm, dy_hbm, out_ref,
        x_vmem, dy_vmem, recv_buf,
        load_sems, send_sems, recv_sems,
    ):
        my_i = lax.axis_index("i")

        barrier_sem = pltpu.get_barrier_semaphore()
        for off in range(1, N_DEV):
            peer = lax.rem(my_i + off, N_DEV)
            pl.semaphore_signal(
                barrier_sem, inc=1,
                device_id=(peer,), device_id_type=pl.DeviceIdType.MESH,
            )

        x_load = pltpu.make_async_copy(x_hbm, x_vmem, load_sems.at[0])
        dy_load = pltpu.make_async_copy(dy_hbm, dy_vmem, load_sems.at[1])
        x_load.start()
        dy_load.start()

        x_load.wait()
        dy_load.wait()
        xv = x_vmem[:, :]
        dyv = dy_vmem[:, :]
        sx = jnp.sum(xv, axis=1, keepdims=True)
        sx2 = jnp.sum(xv * xv, axis=1, keepdims=True)
        mu = sx * (1.0 / d)
        var = sx2 * (1.0 / d) - mu * mu
        rstd = lax.rsqrt(var + 1e-5)
        recv_buf[my_i, :, :] = jnp.stack(
            [jnp.sum(dyv * ((xv - mu) * rstd), axis=0), jnp.sum(dyv, axis=0)],
            axis=0,
        )

        pl.semaphore_wait(barrier_sem, N_DEV - 1)

        sends = []
        for off in range(1, N_DEV):
            peer = lax.rem(my_i + off, N_DEV)
            rdma = pltpu.make_async_remote_copy(
                src_ref=recv_buf.at[my_i],
                dst_ref=recv_buf.at[my_i],
                send_sem=send_sems.at[off - 1],
                recv_sem=recv_sems.at[my_i],
                device_id=(peer,),
                device_id_type=pl.DeviceIdType.MESH,
            )
            rdma.start()
            sends.append(rdma)

        for off in range(1, N_DEV):
            src = lax.rem(my_i + off, N_DEV)
            recv = pltpu.make_async_remote_copy(
                src_ref=recv_buf.at[src],
                dst_ref=recv_buf.at[src],
                send_sem=send_sems.at[off - 1],
                recv_sem=recv_sems.at[src],
                device_id=(src,),
                device_id_type=pl.DeviceIdType.MESH,
            )
            recv.wait_recv()

        out_ref[:, :] = jnp.sum(recv_buf[:, :, :], axis=0)

        for rdma in sends:
            rdma.wait_send()

    return pl.pallas_call(
        body,
        out_shape=jax.ShapeDtypeStruct((2, d), jnp.float32),
        in_specs=[
            pl.BlockSpec(memory_space=pl.ANY),
            pl.BlockSpec(memory_space=pl.ANY),
        ],
        out_specs=pl.BlockSpec(memory_space=pltpu.VMEM),
        scratch_shapes=[
            pltpu.VMEM((m, d), jnp.float32),
            pltpu.VMEM((m, d), jnp.float32),
            pltpu.VMEM((N_DEV, 2, d), jnp.float32),
            pltpu.SemaphoreType.DMA((2,)),
            pltpu.SemaphoreType.DMA((N_DEV - 1,)),
            pltpu.SemaphoreType.DMA((N_DEV,)),
        ],
        compiler_params=pltpu.CompilerParams(collective_id=0),
    )(x, dy)


# device time: 11473 ns/iter; 1.0056x vs baseline; 1.0056x over previous
import jax
import jax.numpy as jnp
from jax import lax
from jax.experimental import pallas as pl
from jax.experimental.pallas import tpu as pltpu

N_DEV = 16


def kernel(x, dy, gamma):
    del gamma
    m, d = x.shape

    def body(x_ref, dy_ref, out_ref, recv_buf, send_sems, recv_sems):
        my_i = lax.axis_index("i")

        barrier_sem = pltpu.get_barrier_semaphore()
        for off in range(1, N_DEV):
            peer = lax.rem(my_i + off, N_DEV)
            pl.semaphore_signal(
                barrier_sem, inc=1,
                device_id=(peer,), device_id_type=pl.DeviceIdType.MESH,
            )

        xv = x_ref[:, :]
        dyv = dy_ref[:, :]
        mu = jnp.mean(xv, axis=1, keepdims=True)
        var = jnp.mean((xv - mu) ** 2, axis=1, keepdims=True)
        rstd = lax.rsqrt(var + 1e-5)
        xhat = (xv - mu) * rstd
        recv_buf[my_i, :, :] = jnp.stack(
            [jnp.sum(dyv * xhat, axis=0), jnp.sum(dyv, axis=0)], axis=0
        )

        pl.semaphore_wait(barrier_sem, N_DEV - 1)

        sends = []
        for off in range(1, N_DEV):
            peer = lax.rem(my_i + off, N_DEV)
            rdma = pltpu.make_async_remote_copy(
                src_ref=recv_buf.at[my_i],
                dst_ref=recv_buf.at[my_i],
                send_sem=send_sems.at[off - 1],
                recv_sem=recv_sems.at[my_i],
                device_id=(peer,),
                device_id_type=pl.DeviceIdType.MESH,
            )
            rdma.start()
            sends.append(rdma)

        for off in range(1, N_DEV):
            src = lax.rem(my_i + off, N_DEV)
            recv = pltpu.make_async_remote_copy(
                src_ref=recv_buf.at[src],
                dst_ref=recv_buf.at[src],
                send_sem=send_sems.at[off - 1],
                recv_sem=recv_sems.at[src],
                device_id=(src,),
                device_id_type=pl.DeviceIdType.MESH,
            )
            recv.wait_recv()

        out_ref[:, :] = jnp.sum(recv_buf[:, :, :], axis=0)

        for rdma in sends:
            rdma.wait_send()

    return pl.pallas_call(
        body,
        out_shape=jax.ShapeDtypeStruct((2, d), jnp.float32),
        in_specs=[
            pl.BlockSpec(memory_space=pltpu.VMEM),
            pl.BlockSpec(memory_space=pltpu.VMEM),
        ],
        out_specs=pl.BlockSpec(memory_space=pltpu.VMEM),
        scratch_shapes=[
            pltpu.VMEM((N_DEV, 2, d), jnp.float32),
            pltpu.SemaphoreType.DMA((N_DEV - 1,)),
            pltpu.SemaphoreType.DMA((N_DEV,)),
        ],
        compiler_params=pltpu.CompilerParams(collective_id=0),
    )(x, dy)
